# baseline (speedup 1.0000x reference)
import functools

import numpy as np
import jax
import jax.numpy as jnp
from jax import lax
from jax.experimental import pallas as pl
from jax.experimental.pallas import tpu as pltpu
from jax.experimental.pallas import tpu_sc as plsc

_L = 8192
_N1 = 64
_N2 = 128
_K2 = 72
_KT = _K2 * _N1
_KMAX = 4096
_DT = 128


@functools.lru_cache(maxsize=None)
def _dft_consts():
    n1 = np.arange(_N1, dtype=np.float64)
    k1 = np.arange(_N1, dtype=np.float64)
    a1 = 2.0 * np.pi * np.outer(k1, n1) / _N1
    c64, s64 = np.cos(a1), -np.sin(a1)
    n2 = np.arange(_N2, dtype=np.float64)
    k2 = np.arange(_K2, dtype=np.float64)
    a2 = 2.0 * np.pi * np.outer(k2, n2) / _N2
    c128, s128 = np.cos(a2), -np.sin(a2)
    at = 2.0 * np.pi * np.outer(k1, n2) / _L
    twc, tws = np.cos(at), -np.sin(at)
    cs64 = np.vstack([c64, s64])
    cs128 = np.vstack([c128, s128])
    return tuple(np.asarray(v, dtype=np.float32)
                 for v in (cs64, cs128, twc, tws))


def _fft_mag_body(x_ref, cs64_ref, cs128_ref, twc_ref, tws_ref, m2_ref):
    f32 = jnp.float32
    x = x_ref[0]
    dn1 = (((1,), (0,)), ((), ()))
    dot = functools.partial(lax.dot_general, preferred_element_type=f32,
                            precision=lax.Precision.HIGHEST)
    a2 = dot(cs64_ref[...], x, dn1)
    are, aim = a2[:_N1], a2[_N1:]
    twc = twc_ref[...][:, :, None]
    tws = tws_ref[...][:, :, None]
    zre = are * twc - aim * tws
    zim = are * tws + aim * twc
    dn2 = (((1,), (1,)), ((), ()))
    p = dot(cs128_ref[...], zre, dn2)
    q = dot(cs128_ref[...], zim, dn2)
    xre = p[:_K2] - q[_K2:]
    xim = q[:_K2] + p[_K2:]
    m2 = xre * xre + xim * xim
    kv = (lax.broadcasted_iota(jnp.int32, (_K2, _N1), 0) * _N1
          + lax.broadcasted_iota(jnp.int32, (_K2, _N1), 1))[:, :, None]
    m2_ref[0] = jnp.where((kv >= 1) & (kv <= _KMAX), m2, -1.0)


_NPART = 8
_KP = _KT // _NPART
_SLAB = 128


def _make_sc_topk(B, D):
    nslab = B * D // _SLAB
    ntask = nslab * _NPART
    spb = nslab // B
    mesh = plsc.VectorSubcoreMesh(core_axis_name="c", subcore_axis_name="s")

    @functools.partial(
        pl.kernel,
        mesh=mesh,
        out_type=(
            jax.ShapeDtypeStruct((B, _NPART, 8, D), jnp.float32),
            jax.ShapeDtypeStruct((B, _NPART, 8, D), jnp.int32),
        ),
        scratch_types=[
            pltpu.VMEM((_KP, _SLAB), jnp.float32),
            pltpu.VMEM((8, _SLAB), jnp.float32),
            pltpu.VMEM((8, _SLAB), jnp.int32),
        ],
    )
    def topk(m2_hbm, outv_hbm, outi_hbm, buf, vout, iout):
        wid = lax.axis_index("s") * 2 + lax.axis_index("c")
        zi = jnp.zeros((16,), jnp.int32)
        neg = jnp.full((16,), -2.0, jnp.float32)
        for r in range(ntask // 32):
            task = wid + 32 * r
            slab = task >> 3
            part = task & 7
            b = jnp.where(slab >= spb, 1, 0)
            d0 = pl.multiple_of((slab - b * spb) * _SLAB, _SLAB)
            k0 = pl.multiple_of(part * _KP, _KP)
            pltpu.sync_copy(
                m2_hbm.at[b, pl.ds(k0, _KP), pl.ds(d0, _SLAB)], buf)
            for g in range(8):
                lo = g * 16

                def body(i, c):
                    t1, t2, t3, i1, i2, i3 = c
                    v = buf[i, pl.ds(lo, 16)]
                    iv = jnp.full((16,), k0, jnp.int32) + i
                    g1 = v > t1
                    g2 = v > t2
                    g3 = v > t3
                    t3n = jnp.where(g2, t2, jnp.where(g3, v, t3))
                    i3n = jnp.where(g2, i2, jnp.where(g3, iv, i3))
                    t2n = jnp.where(g1, t1, jnp.where(g2, v, t2))
                    i2n = jnp.where(g1, i1, jnp.where(g2, iv, i2))
                    t1n = jnp.where(g1, v, t1)
                    i1n = jnp.where(g1, iv, i1)
                    return (t1n, t2n, t3n, i1n, i2n, i3n)

                t1, t2, t3, i1, i2, i3 = lax.fori_loop(
                    0, _KP, body, (neg, neg, neg, zi, zi, zi))
                vout[0, pl.ds(lo, 16)] = t1
                vout[1, pl.ds(lo, 16)] = t2
                vout[2, pl.ds(lo, 16)] = t3
                iout[0, pl.ds(lo, 16)] = i1
                iout[1, pl.ds(lo, 16)] = i2
                iout[2, pl.ds(lo, 16)] = i3
            pltpu.sync_copy(vout, outv_hbm.at[b, part, :, pl.ds(d0, _SLAB)])
            pltpu.sync_copy(iout, outi_hbm.at[b, part, :, pl.ds(d0, _SLAB)])

    return topk


def _recon_body(x_ref, vals_ref, idx_ref, seas_ref, trend_ref):
    f32 = jnp.float32
    x = x_ref[0]
    neg = jnp.full(x.shape[-1:], -2.0, f32)
    zi = jnp.zeros(x.shape[-1:], jnp.int32)
    t1, t2, t3 = neg, neg, neg
    i1, i2, i3 = zi, zi, zi
    for p in range(_NPART):
        for j in range(3):
            v = vals_ref[0, p, j]
            iv = idx_ref[0, p, j]
            g1 = v > t1
            g2 = v > t2
            g3 = v > t3
            t3, i3 = (jnp.where(g2, t2, jnp.where(g3, v, t3)),
                      jnp.where(g2, i2, jnp.where(g3, iv, i3)))
            t2, i2 = (jnp.where(g1, t1, jnp.where(g2, v, t2)),
                      jnp.where(g1, i1, jnp.where(g2, iv, i2)))
            t1, i1 = jnp.where(g1, v, t1), jnp.where(g1, iv, i1)
    n1i = lax.broadcasted_iota(jnp.int32, (_N1, 1), 0)
    n2i = lax.broadcasted_iota(jnp.int32, (_N2, 1), 0)
    seas = jnp.zeros_like(x)
    for kj in (i1, i2, i3):
        pa = (n1i * kj[None, :]) & (_N1 - 1)
        aa = pa.astype(f32) * f32(2.0 * np.pi / _N1)
        ca, sa = jnp.cos(aa), jnp.sin(aa)
        pb = (n2i * kj[None, :]) & (_L - 1)
        ab = pb.astype(f32) * f32(2.0 * np.pi / _L)
        cb, sb = jnp.cos(ab), jnp.sin(ab)
        u_n1 = jnp.sum(x * cb[None, :, :], axis=1)
        v_n1 = jnp.sum(x * sb[None, :, :], axis=1)
        re = jnp.sum(ca * u_n1 - sa * v_n1, axis=0)
        im = -jnp.sum(sa * u_n1 + ca * v_n1, axis=0)
        scale = jnp.where(kj == _KMAX, 1.0, 2.0) * (1.0 / _L)
        wre = re * scale
        wim = im * scale
        u = wre[None, :] * ca - wim[None, :] * sa
        v = wre[None, :] * sa + wim[None, :] * ca
        seas = (seas + u[:, None, :] * cb[None, :, :]
                - v[:, None, :] * sb[None, :, :])
    seas_ref[0] = seas
    trend_ref[0] = x - seas


def _fft_mag(xr, interpret=False):
    B, _, _, D = xr.shape
    cs64, cs128, twc, tws = [jnp.asarray(c) for c in _dft_consts()]
    cspec = lambda shape: pl.BlockSpec(shape, lambda b, j: (0, 0))
    return pl.pallas_call(
        _fft_mag_body,
        grid=(B, D // _DT),
        in_specs=[
            pl.BlockSpec((1, _N1, _N2, _DT), lambda b, j: (b, 0, 0, j)),
            cspec((2 * _N1, _N1)), cspec((2 * _K2, _N2)),
            cspec((_N1, _N2)), cspec((_N1, _N2)),
        ],
        out_specs=pl.BlockSpec((1, _K2, _N1, _DT), lambda b, j: (b, 0, 0, j)),
        out_shape=jax.ShapeDtypeStruct((B, _K2, _N1, D), jnp.float32),
        interpret=interpret,
    )(xr, cs64, cs128, twc, tws)


def _recon(xr, pvals, pidx, interpret=False):
    B, _, _, D = xr.shape
    return pl.pallas_call(
        _recon_body,
        grid=(B, D // _DT),
        in_specs=[
            pl.BlockSpec((1, _N1, _N2, _DT), lambda b, j: (b, 0, 0, j)),
            pl.BlockSpec((1, _NPART, 8, _DT), lambda b, j: (b, 0, 0, j)),
            pl.BlockSpec((1, _NPART, 8, _DT), lambda b, j: (b, 0, 0, j)),
        ],
        out_specs=[
            pl.BlockSpec((1, _N1, _N2, _DT), lambda b, j: (b, 0, 0, j)),
            pl.BlockSpec((1, _N1, _N2, _DT), lambda b, j: (b, 0, 0, j)),
        ],
        out_shape=[
            jax.ShapeDtypeStruct((B, _N1, _N2, D), jnp.float32),
            jax.ShapeDtypeStruct((B, _N1, _N2, D), jnp.float32),
        ],
        interpret=interpret,
    )(xr, pvals, pidx)


def kernel(x):
    B, L, D = x.shape
    xr = x.reshape(B, _N1, _N2, D)
    m2 = _fft_mag(xr)
    pvals, pidx = _make_sc_topk(B, D)(m2.reshape(B, _KT, D))
    seas, trend = _recon(xr, pvals, pidx)
    return (seas.reshape(B, L, D), trend.reshape(B, L, D))
